# baseline (speedup 1.0000x reference)
import functools

import jax
import jax.numpy as jnp
from jax import lax
from jax.experimental import pallas as pl
from jax.experimental.pallas import tpu as pltpu
from jax.experimental.pallas import tpu_sc as plsc

N = 10000
F = 128
E = 320000

NC = 2
NS = 16
NW = NC * NS

CH = 128
ROWS = 80
E_PAD = NW * CH * ROWS
N_PAD = 10240
RPT = 632

_mesh = plsc.VectorSubcoreMesh(
    core_axis_name="c", subcore_axis_name="s", num_cores=NC, num_subcores=NS)


@functools.partial(
    pl.kernel,
    out_type=jax.ShapeDtypeStruct((NC, N_PAD), jnp.float32),
    mesh=_mesh,
    scratch_types=[
        pltpu.VMEM((ROWS, CH), jnp.int32),
        pltpu.VMEM((ROWS, CH), jnp.float32),
        pltpu.VMEM_SHARED((N_PAD,), jnp.float32),
        pltpu.SemaphoreType.DMA,
        pltpu.SemaphoreType.DMA,
    ],
    compiler_params=pltpu.CompilerParams(needs_layout_passes=False),
)
def _deg_kernel(dst_hbm, ew_hbm, zvec_hbm, out_hbm, didx_v, ew_v, deg_sh,
                sem_l, sem_s):
    c = lax.axis_index("c")
    s = lax.axis_index("s")
    wid = c * NS + s
    seg = N_PAD // NS
    l1 = pltpu.async_copy(dst_hbm.at[wid], didx_v, sem_l)
    l2 = pltpu.async_copy(ew_hbm.at[wid], ew_v, sem_l)
    pltpu.sync_copy(zvec_hbm.at[pl.ds(s * seg, seg)], deg_sh.at[pl.ds(s * seg, seg)])
    l1.wait()
    l2.wait()
    plsc.subcore_barrier()

    def batch(g, _):
        def fire(j, carry):
            pltpu.async_copy(ew_v.at[g * 16 + j], deg_sh.at[didx_v.at[g * 16 + j]],
                             sem_s, add=True)
            return carry

        lax.fori_loop(0, 16, fire, 0)

        def drain(j, carry):
            pltpu.make_async_copy(ew_v.at[0], deg_sh.at[didx_v.at[0]], sem_s).wait()
            return carry

        lax.fori_loop(0, 16, drain, 0)
        return _

    lax.fori_loop(0, ROWS // 16, batch, 0)
    plsc.subcore_barrier()
    pltpu.sync_copy(deg_sh.at[pl.ds(s * seg, seg)], out_hbm.at[c, pl.ds(s * seg, seg)])


@functools.partial(
    pl.kernel,
    out_type=jax.ShapeDtypeStruct((NC, N, F), jnp.float32),
    mesh=_mesh,
    scratch_types=[
        pltpu.VMEM((N_PAD // F, F), jnp.float32),
        pltpu.VMEM((4, CH), jnp.int32),
        pltpu.VMEM((4, CH), jnp.int32),
        pltpu.VMEM((4, CH), jnp.float32),
        pltpu.VMEM((CH,), jnp.float32),
        pltpu.VMEM((CH, F), jnp.float32),
        pltpu.VMEM((CH, F), jnp.float32),
        pltpu.VMEM_SHARED((N, F), jnp.float32),
        [pltpu.SemaphoreType.DMA] * 4,
        pltpu.SemaphoreType.DMA,
        pltpu.SemaphoreType.DMA,
        pltpu.SemaphoreType.DMA,
        pltpu.SemaphoreType.DMA,
    ],
    compiler_params=pltpu.CompilerParams(needs_layout_passes=False),
)
def _agg_kernel(x_hbm, src_hbm, dst_hbm, ew_hbm, degp_hbm, zmat_hbm, out_hbm,
                dinv_v, src_v, dst_v, ew_v, a_v, rows0, rows1, acc_sh,
                sem_il, sem_g0, sem_g1, sem_s0, sem_s1):
    c = lax.axis_index("c")
    s = lax.axis_index("s")
    wid = c * NS + s

    rows = (rows0, rows1)
    sem_g = (sem_g0, sem_g1)
    sem_s = (sem_s0, sem_s1)

    def iload(slot, row):
        pltpu.async_copy(src_hbm.at[wid, row], src_v.at[slot], sem_il[slot])
        pltpu.async_copy(dst_hbm.at[wid, row], dst_v.at[slot], sem_il[slot])
        pltpu.async_copy(ew_hbm.at[wid, row], ew_v.at[slot], sem_il[slot])

    def il_wait(slot):
        pltpu.make_async_copy(src_hbm.at[wid, 0], src_v.at[slot], sem_il[slot]).wait()
        pltpu.make_async_copy(dst_hbm.at[wid, 0], dst_v.at[slot], sem_il[slot]).wait()
        pltpu.make_async_copy(ew_hbm.at[wid, 0], ew_v.at[slot], sem_il[slot]).wait()

    ld0 = pltpu.async_copy(degp_hbm.at[0], dinv_v, sem_g0)
    ld1 = pltpu.async_copy(degp_hbm.at[1], rows0.at[pl.ds(0, N_PAD // F)], sem_g1)
    iload(0, 0)
    iload(1, 1)
    iload(2, 2)

    @pl.when(s < NS - 1)
    def _():
        pltpu.sync_copy(zmat_hbm.at[pl.ds(s * RPT, RPT)],
                        acc_sh.at[pl.ds(s * RPT, RPT)])

    @pl.when(s == NS - 1)
    def _():
        pltpu.sync_copy(zmat_hbm.at[pl.ds((NS - 1) * RPT, N - (NS - 1) * RPT)],
                        acc_sh.at[pl.ds((NS - 1) * RPT, N - (NS - 1) * RPT)])

    ld0.wait()
    ld1.wait()

    def mk_dinv(i, _):
        j = i // (F // 16)
        t = lax.rem(i, F // 16) * 16
        d = dinv_v[j, pl.ds(t, 16)] + rows0[j, pl.ds(t, 16)] + 1.0
        h = d * 0.5
        y = plsc.bitcast(0x5F3759DF - (plsc.bitcast(d, jnp.int32) >> 1), jnp.float32)
        y = y * (1.5 - h * y * y)
        y = y * (1.5 - h * y * y)
        y = y * (1.5 - h * y * y)
        dinv_v[j, pl.ds(t, 16)] = y
        return _

    lax.fori_loop(0, (N_PAD // F) * (F // 16), mk_dinv, 0)
    plsc.subcore_barrier()

    def g_issue(slot, p):
        pltpu.async_copy(x_hbm.at[src_v.at[slot]], rows[p], sem_g[p])

    def g_wait(p):
        pltpu.make_async_copy(x_hbm.at[src_v.at[0]], rows[p], sem_g[p]).wait()

    def s_issue(slot, p):
        pltpu.async_copy(rows[p], acc_sh.at[dst_v.at[slot]], sem_s[p], add=True)

    def s_wait(p):
        pltpu.make_async_copy(rows[p], acc_sh.at[dst_v.at[0]], sem_s[p]).wait()

    def scale(slot, p):
        buf = rows[p]
        for t in range(CH // 16):
            s16 = src_v[slot, pl.ds(t * 16, 16)]
            dg = plsc.load_gather(dinv_v, [s16 >> 7, s16 & 127])
            a_v[pl.ds(t * 16, 16)] = ew_v[slot, pl.ds(t * 16, 16)] * dg

        def edge(e, carry):
            ab = plsc.load_gather(a_v, [jnp.full((16,), e, jnp.int32)])
            for f in range(F // 16):
                buf[e, pl.ds(f * 16, 16)] = buf[e, pl.ds(f * 16, 16)] * ab
            return carry

        lax.fori_loop(0, CH, edge, 0, unroll=4)

    def step(j, pos, do_swait=True, do_iload=True, do_gnext=True):
        p = pos % 2
        q = 1 - p
        g_wait(p)
        if do_swait:
            s_wait(q)
        if do_iload:
            iload((pos + 3) % 4, j + 3)
        if do_gnext:
            il_wait((pos + 1) % 4)
            g_issue((pos + 1) % 4, q)
        scale(pos, p)
        s_issue(pos, p)

    il_wait(0)
    g_issue(0, 0)
    step(0, 0, do_swait=False)
    step(1, 1)
    step(2, 2)
    step(3, 3)

    def body(b, _):
        j = 4 * b
        step(j, 0)
        step(j + 1, 1)
        step(j + 2, 2)
        step(j + 3, 3)
        return _

    lax.fori_loop(1, (ROWS - 4) // 4, body, 0)
    step(ROWS - 4, 0)
    step(ROWS - 3, 1, do_iload=False)
    step(ROWS - 2, 2, do_iload=False)
    step(ROWS - 1, 3, do_iload=False, do_gnext=False)
    s_wait(1)
    plsc.subcore_barrier()

    @pl.when(s < NS - 1)
    def _():
        pltpu.sync_copy(acc_sh.at[pl.ds(s * RPT, RPT)],
                        out_hbm.at[c, pl.ds(s * RPT, RPT)])

    @pl.when(s == NS - 1)
    def _():
        pltpu.sync_copy(acc_sh.at[pl.ds((NS - 1) * RPT, N - (NS - 1) * RPT)],
                        out_hbm.at[c, pl.ds((NS - 1) * RPT, N - (NS - 1) * RPT)])


def _gates_body(s0_ref, s1_ref, x_ref, d0_ref, d1_ref, h_ref,
                Wz_ref, bz_ref, Lz_ref, lz_ref,
                Wr_ref, br_ref, Lr_ref, lr_ref,
                Wh_ref, bh_ref, Lh_ref, lh_ref,
                Wo_ref, bo_ref, y_ref, hn_ref):
    dinv = lax.rsqrt(d0_ref[...] + d1_ref[...] + 1.0)
    x = x_ref[...]
    h = h_ref[...]
    P = dinv * (s0_ref[...] + s1_ref[...]) + (dinv * dinv) * x

    def gate(W_ref, b_ref, L_ref, lb_ref, left, right):
        Ltop = L_ref[:F, :]
        Lbot = L_ref[F:, :]
        bias = b_ref[...] @ Ltop + lb_ref[...]
        return left @ (W_ref[...] @ Ltop) + right @ Lbot + bias

    Z = jax.nn.sigmoid(gate(Wz_ref, bz_ref, Lz_ref, lz_ref, P, h))
    R = jax.nn.sigmoid(gate(Wr_ref, br_ref, Lr_ref, lr_ref, P, h))
    Ht = jnp.tanh(gate(Wh_ref, bh_ref, Lh_ref, lh_ref, P, h * R))
    Hn = Z * h + (1.0 - Z) * Ht
    y = jnp.where(Hn > 0, Hn, jnp.exp(Hn) - 1.0) @ Wo_ref[...] + bo_ref[...]
    y_ref[...] = y
    hn_ref[...] = Hn


def kernel(x, edge_index, edge_weight, prev_hidden_state,
           W_z, b_z, L_z, lb_z, W_r, b_r, L_r, lb_r, W_h, b_h, L_h, lb_h,
           W_out, b_out):
    src = edge_index[0]
    dst = edge_index[1]
    pad = E_PAD - E
    pidx = (jnp.arange(pad, dtype=jnp.int32) * 37) % N
    src2d = jnp.concatenate([src, pidx]).reshape(NW, ROWS, CH)
    dst2d = jnp.concatenate([dst, pidx]).reshape(NW, ROWS, CH)
    ew2d = jnp.concatenate(
        [edge_weight, jnp.zeros((pad,), jnp.float32)]).reshape(NW, ROWS, CH)
    zvec = jnp.zeros((N_PAD,), jnp.float32)
    zmat = jnp.zeros((N, F), jnp.float32)

    deg_p = _deg_kernel(dst2d, ew2d, zvec)

    S = _agg_kernel(x, src2d, dst2d, ew2d,
                    deg_p.reshape(NC, N_PAD // F, F), zmat)

    BLK = 1000
    grid = (N // BLK,)
    row_spec = pl.BlockSpec((BLK, F), lambda i: (i, 0))
    col_spec = pl.BlockSpec((BLK, 1), lambda i: (i, 0))

    def full(shape):
        return pl.BlockSpec(shape, lambda i: tuple(0 for _ in shape))

    w_specs = []
    for _ in range(3):
        w_specs += [full((F, F)), full((1, F)), full((2 * F, F)), full((1, F))]
    w_specs += [full((F, F)), full((1, F))]

    y, Hn = pl.pallas_call(
        _gates_body,
        grid=grid,
        in_specs=[row_spec, row_spec, row_spec, col_spec, col_spec, row_spec]
                 + w_specs,
        out_specs=[row_spec, row_spec],
        out_shape=[jax.ShapeDtypeStruct((N, F), jnp.float32),
                   jax.ShapeDtypeStruct((N, F), jnp.float32)],
    )(S[0], S[1], x, deg_p[0, :N].reshape(N, 1), deg_p[1, :N].reshape(N, 1),
      prev_hidden_state,
      W_z, b_z.reshape(1, F), L_z, lb_z.reshape(1, F),
      W_r, b_r.reshape(1, F), L_r, lb_r.reshape(1, F),
      W_h, b_h.reshape(1, F), L_h, lb_h.reshape(1, F),
      W_out, b_out.reshape(1, F))
    return (y, Hn)


# speedup vs baseline: 1.0587x; 1.0587x over previous
import functools

import jax
import jax.numpy as jnp
from jax import lax
from jax.experimental import pallas as pl
from jax.experimental.pallas import tpu as pltpu
from jax.experimental.pallas import tpu_sc as plsc

N = 10000
F = 128
E = 320000

NC = 2
NS = 16
NW = NC * NS

CH = 128
ROWS = 80
E_PAD = NW * CH * ROWS
N_PAD = 10240
RPT = 632

_mesh = plsc.VectorSubcoreMesh(
    core_axis_name="c", subcore_axis_name="s", num_cores=NC, num_subcores=NS)


@functools.partial(
    pl.kernel,
    out_type=jax.ShapeDtypeStruct((NC, N_PAD), jnp.float32),
    mesh=_mesh,
    scratch_types=[
        pltpu.VMEM((ROWS, CH), jnp.int32),
        pltpu.VMEM((ROWS, CH), jnp.float32),
        pltpu.VMEM_SHARED((N_PAD,), jnp.float32),
        pltpu.SemaphoreType.DMA,
        pltpu.SemaphoreType.DMA,
    ],
    compiler_params=pltpu.CompilerParams(needs_layout_passes=False),
)
def _deg_kernel(dst_hbm, ew_hbm, zvec_hbm, out_hbm, didx_v, ew_v, deg_sh,
                sem_l, sem_s):
    c = lax.axis_index("c")
    s = lax.axis_index("s")
    wid = c * NS + s
    seg = N_PAD // NS
    l1 = pltpu.async_copy(dst_hbm.at[wid], didx_v, sem_l)
    l2 = pltpu.async_copy(ew_hbm.at[wid], ew_v, sem_l)
    pltpu.sync_copy(zvec_hbm.at[pl.ds(s * seg, seg)], deg_sh.at[pl.ds(s * seg, seg)])
    l1.wait()
    l2.wait()
    plsc.subcore_barrier()

    def batch(g, _):
        def fire(j, carry):
            pltpu.async_copy(ew_v.at[g * 16 + j], deg_sh.at[didx_v.at[g * 16 + j]],
                             sem_s, add=True)
            return carry

        lax.fori_loop(0, 16, fire, 0)

        def drain(j, carry):
            pltpu.make_async_copy(ew_v.at[0], deg_sh.at[didx_v.at[0]], sem_s).wait()
            return carry

        lax.fori_loop(0, 16, drain, 0)
        return _

    lax.fori_loop(0, ROWS // 16, batch, 0)
    plsc.subcore_barrier()
    pltpu.sync_copy(deg_sh.at[pl.ds(s * seg, seg)], out_hbm.at[c, pl.ds(s * seg, seg)])


@functools.partial(
    pl.kernel,
    out_type=jax.ShapeDtypeStruct((NC, N, F), jnp.float32),
    mesh=_mesh,
    scratch_types=[
        pltpu.VMEM((N_PAD,), jnp.float32),
        pltpu.VMEM((640,), jnp.float32),
        pltpu.VMEM((4, CH), jnp.int32),
        pltpu.VMEM((4, CH), jnp.int32),
        pltpu.VMEM((4, CH), jnp.float32),
        pltpu.VMEM((CH,), jnp.float32),
        pltpu.VMEM((CH, F), jnp.float32),
        pltpu.VMEM((CH, F), jnp.float32),
        pltpu.VMEM_SHARED((N, F), jnp.float32),
        pltpu.VMEM_SHARED((N_PAD,), jnp.float32),
        [pltpu.SemaphoreType.DMA] * 4,
        pltpu.SemaphoreType.DMA,
        pltpu.SemaphoreType.DMA,
        pltpu.SemaphoreType.DMA,
        pltpu.SemaphoreType.DMA,
    ],
    compiler_params=pltpu.CompilerParams(needs_layout_passes=False),
)
def _agg_kernel(x_hbm, src_hbm, dst_hbm, ew_hbm, degp_hbm, zmat_hbm, out_hbm,
                dinv_v, dtmp_v, src_v, dst_v, ew_v, a_v, rows0, rows1, acc_sh,
                dinv_sh, sem_il, sem_g0, sem_g1, sem_s0, sem_s1):
    c = lax.axis_index("c")
    s = lax.axis_index("s")
    wid = c * NS + s

    rows = (rows0, rows1)
    sem_g = (sem_g0, sem_g1)
    sem_s = (sem_s0, sem_s1)

    def iload(slot, row):
        pltpu.async_copy(src_hbm.at[wid, row], src_v.at[slot], sem_il[slot])
        pltpu.async_copy(dst_hbm.at[wid, row], dst_v.at[slot], sem_il[slot])
        pltpu.async_copy(ew_hbm.at[wid, row], ew_v.at[slot], sem_il[slot])

    def il_wait(slot):
        pltpu.make_async_copy(src_hbm.at[wid, 0], src_v.at[slot], sem_il[slot]).wait()
        pltpu.make_async_copy(dst_hbm.at[wid, 0], dst_v.at[slot], sem_il[slot]).wait()
        pltpu.make_async_copy(ew_hbm.at[wid, 0], ew_v.at[slot], sem_il[slot]).wait()

    ld0 = pltpu.async_copy(degp_hbm.at[0, s], rows0.at[pl.ds(0, 5)], sem_g0)
    ld1 = pltpu.async_copy(degp_hbm.at[1, s], rows1.at[pl.ds(0, 5)], sem_g1)
    iload(0, 0)
    iload(1, 1)
    iload(2, 2)

    @pl.when(s < NS - 1)
    def _():
        pltpu.sync_copy(zmat_hbm.at[pl.ds(s * RPT, RPT)],
                        acc_sh.at[pl.ds(s * RPT, RPT)])

    @pl.when(s == NS - 1)
    def _():
        pltpu.sync_copy(zmat_hbm.at[pl.ds((NS - 1) * RPT, N - (NS - 1) * RPT)],
                        acc_sh.at[pl.ds((NS - 1) * RPT, N - (NS - 1) * RPT)])

    ld0.wait()
    ld1.wait()

    def mk_dinv(i, _):
        j = i // (F // 16)
        t = lax.rem(i, F // 16) * 16
        d = rows0[j, pl.ds(t, 16)] + rows1[j, pl.ds(t, 16)] + 1.0
        h = d * 0.5
        y = plsc.bitcast(0x5F3759DF - (plsc.bitcast(d, jnp.int32) >> 1), jnp.float32)
        y = y * (1.5 - h * y * y)
        y = y * (1.5 - h * y * y)
        y = y * (1.5 - h * y * y)
        dtmp_v[pl.ds(i * 16, 16)] = y
        return _

    lax.fori_loop(0, 5 * (F // 16), mk_dinv, 0)
    pltpu.sync_copy(dtmp_v, dinv_sh.at[pl.ds(s * 640, 640)])
    plsc.subcore_barrier()
    pltpu.sync_copy(dinv_sh, dinv_v)

    def g_issue(slot, p):
        pltpu.async_copy(x_hbm.at[src_v.at[slot]], rows[p], sem_g[p])

    def g_wait(p):
        pltpu.make_async_copy(x_hbm.at[src_v.at[0]], rows[p], sem_g[p]).wait()

    def s_issue(slot, p):
        pltpu.async_copy(rows[p], acc_sh.at[dst_v.at[slot]], sem_s[p], add=True)

    def s_wait(p):
        pltpu.make_async_copy(rows[p], acc_sh.at[dst_v.at[0]], sem_s[p]).wait()

    def scale(slot, p):
        buf = rows[p]
        for t in range(CH // 16):
            s16 = src_v[slot, pl.ds(t * 16, 16)]
            dg = plsc.load_gather(dinv_v, [s16])
            a_v[pl.ds(t * 16, 16)] = ew_v[slot, pl.ds(t * 16, 16)] * dg

        def edge(e, carry):
            ab = plsc.load_gather(a_v, [jnp.full((16,), e, jnp.int32)])
            for f in range(F // 16):
                buf[e, pl.ds(f * 16, 16)] = buf[e, pl.ds(f * 16, 16)] * ab
            return carry

        lax.fori_loop(0, CH, edge, 0, unroll=4)

    def step(j, pos, do_swait=True, do_iload=True, do_gnext=True):
        p = pos % 2
        q = 1 - p
        g_wait(p)
        if do_swait:
            s_wait(q)
        if do_iload:
            iload((pos + 3) % 4, j + 3)
        if do_gnext:
            il_wait((pos + 1) % 4)
            g_issue((pos + 1) % 4, q)
        scale(pos, p)
        s_issue(pos, p)

    il_wait(0)
    g_issue(0, 0)
    step(0, 0, do_swait=False)
    step(1, 1)
    step(2, 2)
    step(3, 3)

    def body(b, _):
        j = 4 * b
        step(j, 0)
        step(j + 1, 1)
        step(j + 2, 2)
        step(j + 3, 3)
        return _

    lax.fori_loop(1, (ROWS - 4) // 4, body, 0)
    step(ROWS - 4, 0)
    step(ROWS - 3, 1, do_iload=False)
    step(ROWS - 2, 2, do_iload=False)
    step(ROWS - 1, 3, do_iload=False, do_gnext=False)
    s_wait(1)
    plsc.subcore_barrier()

    @pl.when(s < NS - 1)
    def _():
        pltpu.sync_copy(acc_sh.at[pl.ds(s * RPT, RPT)],
                        out_hbm.at[c, pl.ds(s * RPT, RPT)])

    @pl.when(s == NS - 1)
    def _():
        pltpu.sync_copy(acc_sh.at[pl.ds((NS - 1) * RPT, N - (NS - 1) * RPT)],
                        out_hbm.at[c, pl.ds((NS - 1) * RPT, N - (NS - 1) * RPT)])


def _pre_body(h_ref, Wz_ref, bz_ref, Lz_ref, lz_ref,
              Wr_ref, br_ref, Lr_ref, lr_ref,
              Wh_ref, bh_ref, Lh_ref, lh_ref,
              gz_ref, gr_ref, mz_ref, mr_ref, mh_ref, chb_ref):
    h = h_ref[...]
    mz = Wz_ref[...] @ Lz_ref[:F, :]
    mr = Wr_ref[...] @ Lr_ref[:F, :]
    mh = Wh_ref[...] @ Lh_ref[:F, :]
    gz_ref[...] = h @ Lz_ref[F:, :] + bz_ref[...] @ Lz_ref[:F, :] + lz_ref[...]
    gr_ref[...] = h @ Lr_ref[F:, :] + br_ref[...] @ Lr_ref[:F, :] + lr_ref[...]
    mz_ref[...] = mz
    mr_ref[...] = mr
    mh_ref[...] = mh
    chb_ref[...] = bh_ref[...] @ Lh_ref[:F, :] + lh_ref[...]


def _gates_body(s0_ref, s1_ref, x_ref, d0_ref, d1_ref, h_ref,
                gz_ref, gr_ref, mz_ref, mr_ref, mh_ref, chb_ref,
                Lh_ref, Wo_ref, bo_ref, y_ref, hn_ref):
    dinv = lax.rsqrt(d0_ref[...] + d1_ref[...] + 1.0)
    x = x_ref[...]
    h = h_ref[...]
    P = dinv * (s0_ref[...] + s1_ref[...]) + (dinv * dinv) * x
    Z = jax.nn.sigmoid(P @ mz_ref[...] + gz_ref[...])
    R = jax.nn.sigmoid(P @ mr_ref[...] + gr_ref[...])
    Ht = jnp.tanh(P @ mh_ref[...] + (h * R) @ Lh_ref[F:, :] + chb_ref[...])
    Hn = Z * h + (1.0 - Z) * Ht
    y = jnp.where(Hn > 0, Hn, jnp.exp(Hn) - 1.0) @ Wo_ref[...] + bo_ref[...]
    y_ref[...] = y
    hn_ref[...] = Hn


def kernel(x, edge_index, edge_weight, prev_hidden_state,
           W_z, b_z, L_z, lb_z, W_r, b_r, L_r, lb_r, W_h, b_h, L_h, lb_h,
           W_out, b_out):
    src = edge_index[0]
    dst = edge_index[1]
    pad = E_PAD - E
    pidx = (jnp.arange(pad, dtype=jnp.int32) * 37) % N
    src2d = jnp.concatenate([src, pidx]).reshape(NW, ROWS, CH)
    dst2d = jnp.concatenate([dst, pidx]).reshape(NW, ROWS, CH)
    ew2d = jnp.concatenate(
        [edge_weight, jnp.zeros((pad,), jnp.float32)]).reshape(NW, ROWS, CH)
    zvec = jnp.zeros((N_PAD,), jnp.float32)
    zmat = jnp.zeros((N, F), jnp.float32)

    deg_p = _deg_kernel(dst2d, ew2d, zvec)

    S = _agg_kernel(x, src2d, dst2d, ew2d,
                    deg_p.reshape(NC, NS, N_PAD // F // NS, F), zmat)

    BLK = 1000
    grid = (N // BLK,)
    row_spec = pl.BlockSpec((BLK, F), lambda i: (i, 0))
    col_spec = pl.BlockSpec((BLK, 1), lambda i: (i, 0))

    def full(shape):
        return pl.BlockSpec(shape, lambda i: tuple(0 for _ in shape))

    pre_in = [row_spec]
    for _ in range(3):
        pre_in += [full((F, F)), full((1, F)), full((2 * F, F)), full((1, F))]
    Gz, Gr, Mz, Mr, Mh, chb = pl.pallas_call(
        _pre_body,
        grid=grid,
        in_specs=pre_in,
        out_specs=[row_spec, row_spec, full((F, F)), full((F, F)), full((F, F)),
                   full((1, F))],
        out_shape=[jax.ShapeDtypeStruct((N, F), jnp.float32),
                   jax.ShapeDtypeStruct((N, F), jnp.float32),
                   jax.ShapeDtypeStruct((F, F), jnp.float32),
                   jax.ShapeDtypeStruct((F, F), jnp.float32),
                   jax.ShapeDtypeStruct((F, F), jnp.float32),
                   jax.ShapeDtypeStruct((1, F), jnp.float32)],
    )(prev_hidden_state,
      W_z, b_z.reshape(1, F), L_z, lb_z.reshape(1, F),
      W_r, b_r.reshape(1, F), L_r, lb_r.reshape(1, F),
      W_h, b_h.reshape(1, F), L_h, lb_h.reshape(1, F))

    y, Hn = pl.pallas_call(
        _gates_body,
        grid=grid,
        in_specs=[row_spec, row_spec, row_spec, col_spec, col_spec, row_spec,
                  row_spec, row_spec, full((F, F)), full((F, F)), full((F, F)),
                  full((1, F)), full((2 * F, F)), full((F, F)), full((1, F))],
        out_specs=[row_spec, row_spec],
        out_shape=[jax.ShapeDtypeStruct((N, F), jnp.float32),
                   jax.ShapeDtypeStruct((N, F), jnp.float32)],
    )(S[0], S[1], x, deg_p[0, :N].reshape(N, 1), deg_p[1, :N].reshape(N, 1),
      prev_hidden_state, Gz, Gr, Mz, Mr, Mh, chb, L_h, W_out,
      b_out.reshape(1, F))
    return (y, Hn)
